# Initial kernel scaffold; baseline (speedup 1.0000x reference)
import jax
import jax.numpy as jnp
from jax.experimental import pallas as pl


def kernel(x, W_enc, b_enc, W_dec, b_dec):
    raise NotImplementedError("write your pallas kernel here")


# speedup vs baseline: 9.8933x; 9.8933x over previous
import jax
import jax.numpy as jnp
from jax import lax
from jax.experimental import pallas as pl

K_TOPK = 64


def _f32_to_key(bits):
    return jnp.where(bits < 0, bits ^ 0x7FFFFFFF, bits)


def _enc_body(x_ref, w_ref, b_ref, o_ref):
    xb = x_ref[...].astype(jnp.bfloat16)
    wb = w_ref[...].astype(jnp.bfloat16)
    acc = jnp.dot(xb, wb, preferred_element_type=jnp.float32)
    o_ref[...] = acc + b_ref[...]


def _mask_body(zp_ref, o_ref, *, bm, l, chunk, k_top):
    ncc = l // chunk
    for c in range(ncc):
        sl = pl.ds(c * chunk, chunk)
        bits = lax.bitcast_convert_type(zp_ref[:, sl], jnp.int32)
        key = _f32_to_key(bits)
        o_ref[:, sl] = lax.bitcast_convert_type(key, jnp.float32)

    def step(i, t):
        cand = t + (jnp.int32(1) << (31 - i))
        cnt = jnp.zeros((bm, 1), jnp.int32)
        for c in range(ncc):
            sl = pl.ds(c * chunk, chunk)
            key = lax.bitcast_convert_type(o_ref[:, sl], jnp.int32)
            cnt = cnt + jnp.sum((key >= cand).astype(jnp.int32), axis=1,
                                keepdims=True)
        return jnp.where(cnt >= k_top, cand, t)

    t = lax.fori_loop(0, 32, step, jnp.full((bm, 1), -(2**31), jnp.int32))

    for c in range(ncc):
        sl = pl.ds(c * chunk, chunk)
        key = lax.bitcast_convert_type(o_ref[:, sl], jnp.int32)
        val = lax.bitcast_convert_type(_f32_to_key(key), jnp.float32)
        o_ref[:, sl] = jnp.where(key >= t, val, 0.0)


def _dec_body(z_ref, w_ref, b_ref, o_ref):
    k = pl.program_id(1)
    zb = z_ref[...].astype(jnp.bfloat16)
    wb = w_ref[...].astype(jnp.bfloat16)
    part = jnp.dot(zb, wb, preferred_element_type=jnp.float32)

    @pl.when(k == 0)
    def _():
        o_ref[...] = part + b_ref[...]

    @pl.when(k > 0)
    def _():
        o_ref[...] = o_ref[...] + part


def _impl(x, w_enc, b_enc, w_dec, b_dec, interpret=False):
    b, d = x.shape
    l = w_enc.shape[1]

    bm_a = min(512, b)
    bl_a = min(2048, l)
    z_pre = pl.pallas_call(
        _enc_body,
        grid=(b // bm_a, l // bl_a),
        in_specs=[
            pl.BlockSpec((bm_a, d), lambda i, j: (i, 0)),
            pl.BlockSpec((d, bl_a), lambda i, j: (0, j)),
            pl.BlockSpec((1, bl_a), lambda i, j: (0, j)),
        ],
        out_specs=pl.BlockSpec((bm_a, bl_a), lambda i, j: (i, j)),
        out_shape=jax.ShapeDtypeStruct((b, l), jnp.float32),
        interpret=interpret,
    )(x, w_enc, b_enc.reshape(1, l))

    bm_b = min(128, b)
    chunk = min(1024, l)
    import functools
    z = pl.pallas_call(
        functools.partial(_mask_body, bm=bm_b, l=l, chunk=chunk,
                          k_top=K_TOPK),
        grid=(b // bm_b,),
        in_specs=[pl.BlockSpec((bm_b, l), lambda i: (i, 0))],
        out_specs=pl.BlockSpec((bm_b, l), lambda i: (i, 0)),
        out_shape=jax.ShapeDtypeStruct((b, l), jnp.float32),
        interpret=interpret,
    )(z_pre)

    bm_c = min(512, b)
    bk_c = min(2048, l)
    recon = pl.pallas_call(
        _dec_body,
        grid=(b // bm_c, l // bk_c),
        in_specs=[
            pl.BlockSpec((bm_c, bk_c), lambda i, j: (i, j)),
            pl.BlockSpec((bk_c, d), lambda i, j: (j, 0)),
            pl.BlockSpec((1, d), lambda i, j: (0, 0)),
        ],
        out_specs=pl.BlockSpec((bm_c, d), lambda i, j: (i, 0)),
        out_shape=jax.ShapeDtypeStruct((b, d), jnp.float32),
        interpret=interpret,
    )(z, w_dec, b_dec.reshape(1, d))

    return recon, z


def kernel(x, W_enc, b_enc, W_dec, b_dec):
    return _impl(x, W_enc, b_enc, W_dec, b_dec)
